# baseline (speedup 1.0000x reference)
import functools

import jax
import jax.numpy as jnp
from jax import lax
from jax.experimental import pallas as pl
from jax.experimental.pallas import tpu as pltpu
from jax.experimental.pallas import tpu_sc as plsc

EMBED_DIM = 32
BATCH = 16384
HIST = 200
IW = 128
NW = 32
COLS_PER_W = BATCH // NW
NBLK = COLS_PER_W // IW
NB = COLS_PER_W


def _make_gather():
    mesh = plsc.VectorSubcoreMesh(core_axis_name="c", subcore_axis_name="s")

    @functools.partial(
        pl.kernel,
        mesh=mesh,
        compiler_params=pltpu.CompilerParams(
            use_tc_tiling_on_sc=False, needs_layout_passes=False
        ),
        out_type=jax.ShapeDtypeStruct((HIST, EMBED_DIM, BATCH), jnp.float32),
        scratch_types=[
            pltpu.VMEM((NBLK, IW), jnp.int32),
            pltpu.VMEM((NB, EMBED_DIM), jnp.float32),
            pltpu.VMEM((EMBED_DIM, NB), jnp.float32),
            pltpu.SemaphoreType.DMA,
        ],
    )
    def gather_kernel(tids_hbm, table_hbm, out_hbm, idx_v, g_v, gt_v, sem):
        wid = lax.axis_index("s") * 2 + lax.axis_index("c")
        bw = wid * COLS_PER_W
        bw128 = wid * NBLK

        def body(h, carry):
            pltpu.sync_copy(tids_hbm.at[h, pl.ds(bw128, NBLK)], idx_v)
            copies = []
            for j in range(NBLK):
                copies.append(
                    pltpu.async_copy(
                        table_hbm.at[idx_v.at[j]],
                        g_v.at[pl.ds(j * IW, IW)],
                        sem,
                    )
                )
            for cp in copies:
                cp.wait()

            def tbody(d, cc):
                col = jnp.full((16,), d, dtype=jnp.int32)
                for j in range(NB // 16):
                    rows = j * 16 + lax.iota(jnp.int32, 16)
                    vec = plsc.load_gather(g_v, [rows, col])
                    gt_v[d, pl.ds(j * 16, 16)] = vec
                return cc

            lax.fori_loop(0, EMBED_DIM, tbody, 0)
            pltpu.sync_copy(gt_v, out_hbm.at[h, :, pl.ds(bw, NB)])
            return carry

        lax.fori_loop(0, HIST, body, 0)

    return gather_kernel


_gather = _make_gather()


def kernel(token_ids, embed_weight):
    tids3 = token_ids.astype(jnp.int32).T.reshape(HIST, BATCH // IW, IW)
    out = _gather(tids3, embed_weight)
    return jnp.transpose(out, (2, 0, 1))


# speedup vs baseline: 2.5533x; 2.5533x over previous
import functools

import jax
import jax.numpy as jnp
from jax import lax
from jax.experimental import pallas as pl
from jax.experimental.pallas import tpu as pltpu
from jax.experimental.pallas import tpu_sc as plsc

EMBED_DIM = 32
BATCH = 16384
HIST = 200
IW = 128
NW = 32
COLS_PER_W = BATCH // NW
NBLK = COLS_PER_W // IW
NB = COLS_PER_W
GT_STRIDE = NB + 1
N_PAIR = HIST // 2


def _make_gather():
    mesh = plsc.VectorSubcoreMesh(core_axis_name="c", subcore_axis_name="s")

    @functools.partial(
        pl.kernel,
        mesh=mesh,
        compiler_params=pltpu.CompilerParams(
            use_tc_tiling_on_sc=False, needs_layout_passes=False
        ),
        out_type=jax.ShapeDtypeStruct((HIST, EMBED_DIM, BATCH), jnp.float32),
        scratch_types=[
            pltpu.VMEM((NBLK, IW), jnp.int32),
            pltpu.VMEM((NBLK, IW), jnp.int32),
            pltpu.VMEM((NB, EMBED_DIM), jnp.float32),
            pltpu.VMEM((NB, EMBED_DIM), jnp.float32),
            pltpu.VMEM((EMBED_DIM, GT_STRIDE), jnp.float32),
            pltpu.VMEM((EMBED_DIM, GT_STRIDE), jnp.float32),
            pltpu.SemaphoreType.DMA,
            pltpu.SemaphoreType.DMA,
            pltpu.SemaphoreType.DMA,
            pltpu.SemaphoreType.DMA,
        ],
    )
    def gather_kernel(tids_hbm, table_hbm, out_hbm,
                      idx0, idx1, g0, g1, gt0, gt1, sg0, sg1, ss0, ss1):
        idxs = (idx0, idx1)
        gs = (g0, g1)
        gts = (gt0, gt1)
        sgs = (sg0, sg1)
        sss = (ss0, ss1)
        wid = lax.axis_index("s") * 2 + lax.axis_index("c")
        bw = wid * COLS_PER_W
        bw128 = wid * NBLK

        iota = lax.iota(jnp.int32, 16)
        row_lo = iota
        row_hi = iota + 16

        def load_and_fire(h, b):
            pltpu.sync_copy(tids_hbm.at[h, pl.ds(bw128, NBLK)], idxs[b])
            for j in range(NBLK):
                pltpu.async_copy(
                    table_hbm.at[idxs[b].at[j]],
                    gs[b].at[pl.ds(j * IW, IW)],
                    sgs[b],
                )

        def drain_gather(b):
            pltpu.make_async_copy(
                table_hbm.at[idxs[b].at[0]], gs[b], sgs[b]
            ).wait()

        def fire_store(h, b):
            pltpu.async_copy(
                gts[b].at[:, pl.ds(0, NB)],
                out_hbm.at[h, :, pl.ds(bw, NB)],
                sss[b],
            )

        def drain_store(b):
            pltpu.make_async_copy(
                gts[b].at[:, pl.ds(0, NB)],
                out_hbm.at[0, :, pl.ds(bw, NB)],
                sss[b],
            ).wait()

        def transpose(b):
            g = gs[b]
            gt = gts[b]

            def tbody(tg, cc):
                t0 = tg * 16
                for tt in range(16):
                    t = t0 + tt
                    colv = jnp.full((16,), t, dtype=jnp.int32)
                    v_lo = g[t, pl.ds(0, 16)]
                    v_hi = g[t, pl.ds(16, 16)]
                    plsc.store_scatter(gt, [row_lo, colv], v_lo)
                    plsc.store_scatter(gt, [row_hi, colv], v_hi)
                return cc

            lax.fori_loop(0, NB // 16, tbody, 0)

        load_and_fire(0, 0)

        def body(ci, carry):
            for b in range(2):
                h = ci * 2 + b

                @pl.when(h + 1 < HIST)
                def _():
                    load_and_fire(h + 1, 1 - b)

                drain_gather(b)

                @pl.when(ci > 0)
                def _():
                    drain_store(b)

                transpose(b)
                fire_store(h, b)
            return carry

        lax.fori_loop(0, N_PAIR, body, 0)
        drain_store(0)
        drain_store(1)

    return gather_kernel


_gather = _make_gather()


def kernel(token_ids, embed_weight):
    tids3 = token_ids.astype(jnp.int32).T.reshape(HIST, BATCH // IW, IW)
    out = _gather(tids3, embed_weight)
    return jnp.transpose(out, (2, 0, 1))


# speedup vs baseline: 3.3051x; 1.2944x over previous
import functools

import jax
import jax.numpy as jnp
from jax import lax
from jax.experimental import pallas as pl
from jax.experimental.pallas import tpu as pltpu
from jax.experimental.pallas import tpu_sc as plsc

EMBED_DIM = 32
BATCH = 16384
HIST = 200
IW = 128
NW = 32
COLS_PER_W = BATCH // NW
NBLK = COLS_PER_W // IW
NB = COLS_PER_W
GT_STRIDE = NB + 1
N_PAIR = HIST // 2


def _make_gather():
    mesh = plsc.VectorSubcoreMesh(core_axis_name="c", subcore_axis_name="s")

    @functools.partial(
        pl.kernel,
        mesh=mesh,
        compiler_params=pltpu.CompilerParams(
            use_tc_tiling_on_sc=False, needs_layout_passes=False
        ),
        out_type=jax.ShapeDtypeStruct((HIST, 4, BATCH // IW, 8, IW), jnp.float32),
        scratch_types=[
            pltpu.VMEM((NBLK, IW), jnp.int32),
            pltpu.VMEM((NBLK, IW), jnp.int32),
            pltpu.VMEM((NB, EMBED_DIM), jnp.float32),
            pltpu.VMEM((NB, EMBED_DIM), jnp.float32),
            pltpu.VMEM((NBLK, 4, 8, IW + 1), jnp.float32),
            pltpu.VMEM((NBLK, 4, 8, IW + 1), jnp.float32),
            pltpu.SemaphoreType.DMA,
            pltpu.SemaphoreType.DMA,
            pltpu.SemaphoreType.DMA,
            pltpu.SemaphoreType.DMA,
        ],
    )
    def gather_kernel(tids_hbm, table_hbm, out_hbm,
                      idx0, idx1, g0, g1, gt0, gt1, sg0, sg1, ss0, ss1):
        idxs = (idx0, idx1)
        gs = (g0, g1)
        gts = (gt0, gt1)
        sgs = (sg0, sg1)
        sss = (ss0, ss1)
        wid = lax.axis_index("s") * 2 + lax.axis_index("c")
        bw = wid * COLS_PER_W
        bw128 = wid * NBLK

        iota = lax.iota(jnp.int32, 16)
        dhi_lo = iota // 8
        dhi_hi = dhi_lo + 2
        dlo_v = iota % 8

        def load_and_fire(h, b):
            pltpu.sync_copy(tids_hbm.at[h, pl.ds(bw128, NBLK)], idxs[b])
            for j in range(NBLK):
                pltpu.async_copy(
                    table_hbm.at[idxs[b].at[j]],
                    gs[b].at[pl.ds(j * IW, IW)],
                    sgs[b],
                )

        def drain_gather(b):
            pltpu.make_async_copy(
                table_hbm.at[idxs[b].at[0]], gs[b], sgs[b]
            ).wait()

        def fire_store(h, b):
            for dhi in range(4):
                pltpu.async_copy(
                    gts[b].at[:, dhi, :, pl.ds(0, IW)],
                    out_hbm.at[h, dhi, pl.ds(bw128, NBLK)],
                    sss[b],
                )

        def drain_store(b):
            for dhi in range(4):
                pltpu.make_async_copy(
                    gts[b].at[:, dhi, :, pl.ds(0, IW)],
                    out_hbm.at[0, dhi, pl.ds(bw128, NBLK)],
                    sss[b],
                ).wait()

        def transpose(b):
            g = gs[b]
            gt = gts[b]

            def tbody(tg, cc):
                t0 = tg * 16
                bhi_v = jnp.full((16,), t0 // IW, dtype=jnp.int32)
                blo_base = t0 % IW
                for tt in range(16):
                    blo_v = jnp.full((16,), blo_base + tt, dtype=jnp.int32)
                    t = t0 + tt
                    v_lo = g[t, pl.ds(0, 16)]
                    v_hi = g[t, pl.ds(16, 16)]
                    plsc.store_scatter(gt, [bhi_v, dhi_lo, dlo_v, blo_v], v_lo)
                    plsc.store_scatter(gt, [bhi_v, dhi_hi, dlo_v, blo_v], v_hi)
                return cc

            lax.fori_loop(0, NB // 16, tbody, 0)

        load_and_fire(0, 0)

        def body(ci, carry):
            for b in range(2):
                h = ci * 2 + b

                @pl.when(h + 1 < HIST)
                def _():
                    load_and_fire(h + 1, 1 - b)

                drain_gather(b)

                @pl.when(ci > 0)
                def _():
                    drain_store(b)

                transpose(b)
                fire_store(h, b)
            return carry

        lax.fori_loop(0, N_PAIR, body, 0)
        drain_store(0)
        drain_store(1)

    return gather_kernel


_gather = _make_gather()


def kernel(token_ids, embed_weight):
    tids3 = token_ids.astype(jnp.int32).T.reshape(HIST, BATCH // IW, IW)
    out = _gather(tids3, embed_weight)
    return jnp.transpose(out, (2, 4, 0, 1, 3)).reshape(BATCH, HIST, EMBED_DIM)


# speedup vs baseline: 3.8607x; 1.1681x over previous
import functools

import jax
import jax.numpy as jnp
from jax import lax
from jax.experimental import pallas as pl
from jax.experimental.pallas import tpu as pltpu
from jax.experimental.pallas import tpu_sc as plsc

EMBED_DIM = 32
BATCH = 16384
HIST = 200
IW = 128
NW = 32
COLS_PER_W = BATCH // NW
NBLK = COLS_PER_W // IW
NB = COLS_PER_W
GT_STRIDE = NB + 1
N_PAIR = HIST // 2


def _make_gather():
    mesh = plsc.VectorSubcoreMesh(core_axis_name="c", subcore_axis_name="s")

    @functools.partial(
        pl.kernel,
        mesh=mesh,
        compiler_params=pltpu.CompilerParams(
            use_tc_tiling_on_sc=False, needs_layout_passes=False
        ),
        out_type=jax.ShapeDtypeStruct((HIST, 4, BATCH // IW, 8, IW), jnp.float32),
        scratch_types=[
            pltpu.VMEM((NBLK, IW), jnp.int32),
            pltpu.VMEM((NBLK, IW), jnp.int32),
            pltpu.VMEM((NB, EMBED_DIM), jnp.float32),
            pltpu.VMEM((NB, EMBED_DIM), jnp.float32),
            pltpu.VMEM((NBLK, 4, 8, IW + 1), jnp.float32),
            pltpu.VMEM((NBLK, 4, 8, IW + 1), jnp.float32),
            pltpu.SemaphoreType.DMA,
            pltpu.SemaphoreType.DMA,
            pltpu.SemaphoreType.DMA,
            pltpu.SemaphoreType.DMA,
            pltpu.SemaphoreType.DMA,
            pltpu.SemaphoreType.DMA,
        ],
    )
    def gather_kernel(tids_hbm, table_hbm, out_hbm,
                      idx0, idx1, g0, g1, gt0, gt1,
                      sg0, sg1, ss0, ss1, si0, si1):
        idxs = (idx0, idx1)
        gs = (g0, g1)
        gts = (gt0, gt1)
        sgs = (sg0, sg1)
        sss = (ss0, ss1)
        sis = (si0, si1)
        wid = lax.axis_index("s") * 2 + lax.axis_index("c")
        bw = wid * COLS_PER_W
        bw128 = wid * NBLK

        iota = lax.iota(jnp.int32, 16)
        dhi_lo = iota // 8
        dhi_hi = dhi_lo + 2
        dlo_v = iota % 8

        def fire_idx(h, b):
            pltpu.async_copy(
                tids_hbm.at[h, pl.ds(bw128, NBLK)], idxs[b], sis[b]
            )

        def wait_idx(b):
            pltpu.make_async_copy(
                tids_hbm.at[0, pl.ds(bw128, NBLK)], idxs[b], sis[b]
            ).wait()

        def fire_gathers(b):
            for j in range(NBLK):
                pltpu.async_copy(
                    table_hbm.at[idxs[b].at[j]],
                    gs[b].at[pl.ds(j * IW, IW)],
                    sgs[b],
                )

        def drain_gather(b):
            pltpu.make_async_copy(
                table_hbm.at[idxs[b].at[0]], gs[b], sgs[b]
            ).wait()

        def fire_store(h, b):
            for dhi in range(4):
                pltpu.async_copy(
                    gts[b].at[:, dhi, :, pl.ds(0, IW)],
                    out_hbm.at[h, dhi, pl.ds(bw128, NBLK)],
                    sss[b],
                )

        def drain_store(b):
            for dhi in range(4):
                pltpu.make_async_copy(
                    gts[b].at[:, dhi, :, pl.ds(0, IW)],
                    out_hbm.at[0, dhi, pl.ds(bw128, NBLK)],
                    sss[b],
                ).wait()

        def transpose(b):
            g = gs[b]
            gt = gts[b]

            def tbody(tg, cc):
                t0 = tg * 16
                bhi_v = jnp.full((16,), t0 // IW, dtype=jnp.int32)
                blo_base = t0 % IW
                for tt in range(16):
                    blo_v = jnp.full((16,), blo_base + tt, dtype=jnp.int32)
                    t = t0 + tt
                    v_lo = g[t, pl.ds(0, 16)]
                    v_hi = g[t, pl.ds(16, 16)]
                    plsc.store_scatter(gt, [bhi_v, dhi_lo, dlo_v, blo_v], v_lo)
                    plsc.store_scatter(gt, [bhi_v, dhi_hi, dlo_v, blo_v], v_hi)
                return cc

            lax.fori_loop(0, NB // 16, tbody, 0)

        fire_idx(0, 0)
        wait_idx(0)
        fire_gathers(0)
        fire_idx(1, 1)

        def body(ci, carry):
            for b in range(2):
                h = ci * 2 + b

                @pl.when(h + 1 < HIST)
                def _():
                    wait_idx(1 - b)
                    fire_gathers(1 - b)

                drain_gather(b)

                @pl.when(h + 2 < HIST)
                def _():
                    fire_idx(h + 2, b)

                @pl.when(ci > 0)
                def _():
                    drain_store(b)

                transpose(b)
                fire_store(h, b)
            return carry

        lax.fori_loop(0, N_PAIR, body, 0)
        drain_store(0)
        drain_store(1)

    return gather_kernel


_gather = _make_gather()


def kernel(token_ids, embed_weight):
    tids3 = token_ids.astype(jnp.int32).T.reshape(HIST, BATCH // IW, IW)
    out = _gather(tids3, embed_weight)
    return jnp.transpose(out, (2, 4, 0, 1, 3)).reshape(BATCH, HIST, EMBED_DIM)


# speedup vs baseline: 3.8621x; 1.0004x over previous
import functools

import jax
import jax.numpy as jnp
from jax import lax
from jax.experimental import pallas as pl
from jax.experimental.pallas import tpu as pltpu
from jax.experimental.pallas import tpu_sc as plsc

EMBED_DIM = 32
BATCH = 16384
HIST = 200
IW = 128
NW = 32
COLS_PER_W = BATCH // NW
NBLK = COLS_PER_W // IW
NB = COLS_PER_W
N_PAIR = HIST // 2


def _make_gather():
    mesh = plsc.VectorSubcoreMesh(core_axis_name="c", subcore_axis_name="s")

    @functools.partial(
        pl.kernel,
        mesh=mesh,
        compiler_params=pltpu.CompilerParams(
            use_tc_tiling_on_sc=False, needs_layout_passes=False
        ),
        out_type=jax.ShapeDtypeStruct((HIST, 4, BATCH // IW, 8, IW), jnp.float32),
        scratch_types=[
            pltpu.VMEM((NBLK, IW), jnp.int32),
            pltpu.VMEM((NBLK, IW), jnp.int32),
            pltpu.VMEM((NB, EMBED_DIM), jnp.float32),
            pltpu.VMEM((NB, EMBED_DIM), jnp.float32),
            pltpu.VMEM((NBLK, 4, 8, IW + 1), jnp.float32),
            pltpu.VMEM((NBLK, 4, 8, IW + 1), jnp.float32),
            pltpu.SemaphoreType.DMA,
            pltpu.SemaphoreType.DMA,
            pltpu.SemaphoreType.DMA,
            pltpu.SemaphoreType.DMA,
            pltpu.SemaphoreType.DMA,
            pltpu.SemaphoreType.DMA,
        ],
    )
    def gather_kernel(tids_hbm, table_hbm, out_hbm,
                      idx0, idx1, g0, g1, gt0, gt1,
                      sg0, sg1, ss0, ss1, si0, si1):
        idxs = (idx0, idx1)
        gs = (g0, g1)
        gts = (gt0, gt1)
        sgs = (sg0, sg1)
        sss = (ss0, ss1)
        sis = (si0, si1)
        wid = lax.axis_index("s") * 2 + lax.axis_index("c")
        bw = wid * COLS_PER_W
        bw128 = wid * NBLK

        iota = lax.iota(jnp.int32, 16)
        dhi_lo = iota // 8
        dhi_hi = dhi_lo + 2
        dlo_v = iota % 8

        def fire_idx(h, b):
            pltpu.async_copy(
                tids_hbm.at[h, pl.ds(bw128, NBLK)], idxs[b], sis[b]
            )

        def wait_idx(b):
            pltpu.make_async_copy(
                tids_hbm.at[0, pl.ds(bw128, NBLK)], idxs[b], sis[b]
            ).wait()

        def fire_gathers(b):
            for j in range(NBLK):
                pltpu.async_copy(
                    table_hbm.at[idxs[b].at[j]],
                    gs[b].at[pl.ds(j * IW, IW)],
                    sgs[b],
                )

        def drain_gather(b):
            pltpu.make_async_copy(
                table_hbm.at[idxs[b].at[0]], gs[b], sgs[b]
            ).wait()

        def fire_store(h, b):
            for dhi in range(4):
                pltpu.async_copy(
                    gts[b].at[:, dhi, :, pl.ds(0, IW)],
                    out_hbm.at[h, dhi, pl.ds(bw128, NBLK)],
                    sss[b],
                )

        def drain_store(b):
            for dhi in range(4):
                pltpu.make_async_copy(
                    gts[b].at[:, dhi, :, pl.ds(0, IW)],
                    out_hbm.at[0, dhi, pl.ds(bw128, NBLK)],
                    sss[b],
                ).wait()

        def transpose(b):
            g = gs[b]
            gt = gts[b]

            def tbody(tg, cc):
                t0 = tg * 16
                bhi_v = jnp.full((16,), t0 // IW, dtype=jnp.int32)
                blo_base = t0 % IW
                for tt in range(16):
                    blo_v = jnp.full((16,), blo_base + tt, dtype=jnp.int32)
                    t = t0 + tt
                    v_lo = g[t, pl.ds(0, 16)]
                    v_hi = g[t, pl.ds(16, 16)]
                    plsc.store_scatter(gt, [bhi_v, dhi_lo, dlo_v, blo_v], v_lo)
                    plsc.store_scatter(gt, [bhi_v, dhi_hi, dlo_v, blo_v], v_hi)
                return cc

            lax.fori_loop(0, NB // 16, tbody, 0)

        fire_idx(0, 0)
        wait_idx(0)
        fire_gathers(0)
        fire_idx(1, 1)

        def body(ci, carry):
            for b in range(2):
                h = ci * 2 + b

                @pl.when(h + 1 < HIST)
                def _():
                    wait_idx(1 - b)
                    fire_gathers(1 - b)

                drain_gather(b)

                @pl.when(h + 2 < HIST)
                def _():
                    fire_idx(h + 2, b)

                @pl.when(ci > 0)
                def _():
                    drain_store(b)

                transpose(b)
                fire_store(h, b)
            return carry

        lax.fori_loop(0, N_PAIR, body, 0)
        drain_store(0)
        drain_store(1)

    return gather_kernel


_gather = _make_gather()


def kernel(token_ids, embed_weight):
    tids3 = token_ids.astype(jnp.int32).T.reshape(HIST, BATCH // IW, IW)
    out = _gather(tids3, embed_weight)
    return jnp.transpose(out, (2, 4, 0, 1, 3)).reshape(BATCH, HIST, EMBED_DIM)
